# baseline (speedup 1.0000x reference)
import functools

import jax
import jax.numpy as jnp
from jax import lax
from jax.experimental import pallas as pl
from jax.experimental.pallas import tpu as pltpu
from jax.experimental.pallas import tpu_sc as plsc

D_MODEL = 64
SCALE_F = 8.0
NUM_CORES = 2
NUM_SUBCORES = 16
NUM_WORKERS = NUM_CORES * NUM_SUBCORES
LANES = 16
CHUNK = 800


def kernel(token_ids, table):
    batch_shape = token_ids.shape
    idx = token_ids.reshape(-1)
    num_ids = idx.shape[0]
    per_worker = num_ids // NUM_WORKERS
    n_chunks = per_worker // CHUNK
    assert per_worker * NUM_WORKERS == num_ids
    assert n_chunks * CHUNK == per_worker

    mesh = plsc.VectorSubcoreMesh(core_axis_name="c", subcore_axis_name="s")

    @functools.partial(
        pl.kernel,
        mesh=mesh,
        out_type=jax.ShapeDtypeStruct((num_ids, D_MODEL), jnp.float32),
        scratch_types=[
            pltpu.VMEM((CHUNK,), jnp.int32),
            pltpu.VMEM((CHUNK, D_MODEL), jnp.float32),
            pltpu.SemaphoreType.DMA,
        ],
        compiler_params=pltpu.CompilerParams(use_tc_tiling_on_sc=False),
    )
    def gather_scale(table_hbm, idx_hbm, out_hbm, idx_v, rows_v, sem):
        wid = lax.axis_index("s") * NUM_CORES + lax.axis_index("c")
        base0 = wid * per_worker

        @pl.loop(0, n_chunks)
        def _(j):
            base = base0 + j * CHUNK
            pltpu.sync_copy(idx_hbm.at[pl.ds(base, CHUNK)], idx_v)
            pltpu.async_copy(table_hbm.at[idx_v], rows_v, sem).wait()

            @pl.loop(0, CHUNK)
            def _(r):
                for c in range(0, D_MODEL, LANES):
                    sl = (r, pl.ds(c, LANES))
                    rows_v.at[sl][...] = rows_v.at[sl][...] * SCALE_F

            pltpu.sync_copy(rows_v, out_hbm.at[pl.ds(base, CHUNK)])

    out = gather_scale(table, idx)
    return out.reshape(*batch_shape, D_MODEL)
